# Initial kernel scaffold; baseline (speedup 1.0000x reference)
import functools

import jax
import jax.numpy as jnp
from jax.experimental import pallas as pl
from jax.experimental.pallas import tpu as pltpu

DIM = 512
HID = 2048
E = 8
K = 2


def _gate_weights(logits):
    T = logits.shape[0]
    col = jax.lax.broadcasted_iota(jnp.int32, (T, E), 1)
    m1 = jnp.max(logits, axis=1, keepdims=True)
    big = jnp.int32(E)
    idx1 = jnp.min(jnp.where(logits == m1, col, big), axis=1, keepdims=True)
    masked = jnp.where(col == idx1, -jnp.inf, logits)
    m2 = jnp.max(masked, axis=1, keepdims=True)
    idx2 = jnp.min(jnp.where(masked == m2, col, big), axis=1, keepdims=True)
    e2 = jnp.exp(m2 - m1)
    p1 = 1.0 / (1.0 + e2)
    p2 = 1.0 - p1
    return jnp.where(col == idx1, p1, jnp.where(col == idx2, p2, 0.0))


def _moe_body(x_ref, gw_ref, w1_ref, w2_ref, o_ref):
    e = pl.program_id(0)
    xb = x_ref[...]
    logits = jax.lax.dot_general(
        xb, gw_ref[...], (((1,), (1,)), ((), ())),
        preferred_element_type=jnp.float32)
    w_full = _gate_weights(logits)
    we = jax.lax.dynamic_slice(w_full, (0, e), (w_full.shape[0], 1))
    w1e = w1_ref[0]
    w2e = w2_ref[0]
    h = jax.lax.dot_general(
        xb, w1e, (((1,), (1,)), ((), ())), preferred_element_type=jnp.float32)
    h = jnp.maximum(h, 0.0)
    y = jax.lax.dot_general(
        h, w2e, (((1,), (1,)), ((), ())), preferred_element_type=jnp.float32)
    contrib = we * y

    @pl.when(e == 0)
    def _():
        o_ref[...] = contrib

    @pl.when(e > 0)
    def _():
        o_ref[...] = o_ref[...] + contrib


@jax.jit
def kernel(x, gate_w, w1, w2):
    B, N, D = x.shape
    T = B * N
    xf = x.reshape(T, D)
    out = pl.pallas_call(
        _moe_body,
        grid=(E,),
        in_specs=[
            pl.BlockSpec((T, D), lambda e: (0, 0)),
            pl.BlockSpec((E, D), lambda e: (0, 0)),
            pl.BlockSpec((1, HID, D), lambda e: (e, 0, 0)),
            pl.BlockSpec((1, D, HID), lambda e: (e, 0, 0)),
        ],
        out_specs=pl.BlockSpec((T, D), lambda e: (0, 0)),
        out_shape=jax.ShapeDtypeStruct((T, D), jnp.float32),
    )(xf, gate_w, w1, w2)
    return out.reshape(B, N, D)


# speedup vs baseline: 1.2449x; 1.2449x over previous
import functools

import jax
import jax.numpy as jnp
from jax.experimental import pallas as pl
from jax.experimental.pallas import tpu as pltpu

DIM = 512
HID = 2048
E = 8
K = 2


def _gate_weights(logits):
    T = logits.shape[0]
    col = jax.lax.broadcasted_iota(jnp.int32, (T, E), 1)
    m1 = jnp.max(logits, axis=1, keepdims=True)
    big = jnp.int32(E)
    idx1 = jnp.min(jnp.where(logits == m1, col, big), axis=1, keepdims=True)
    masked = jnp.where(col == idx1, -jnp.inf, logits)
    m2 = jnp.max(masked, axis=1, keepdims=True)
    idx2 = jnp.min(jnp.where(masked == m2, col, big), axis=1, keepdims=True)
    e2 = jnp.exp(m2 - m1)
    p1 = 1.0 / (1.0 + e2)
    p2 = 1.0 - p1
    return jnp.where(col == idx1, p1, jnp.where(col == idx2, p2, 0.0))


def _moe_body(x_ref, gw_ref, w1_ref, w2_ref, o_ref):
    e = pl.program_id(0)
    xb = x_ref[...]
    logits = jax.lax.dot_general(
        xb, gw_ref[...], (((1,), (1,)), ((), ())),
        preferred_element_type=jnp.float32)
    w_full = _gate_weights(logits)
    col = jax.lax.broadcasted_iota(jnp.int32, w_full.shape, 1)
    we = jnp.sum(jnp.where(col == e, w_full, 0.0), axis=1, keepdims=True)
    w1e = w1_ref[0]
    w2e = w2_ref[0]
    h = jax.lax.dot_general(
        xb, w1e, (((1,), (1,)), ((), ())), preferred_element_type=jnp.float32)
    h = jnp.maximum(h, 0.0)
    y = jax.lax.dot_general(
        h, w2e, (((1,), (1,)), ((), ())), preferred_element_type=jnp.float32)
    contrib = we * y

    @pl.when(e == 0)
    def _():
        o_ref[...] = contrib

    @pl.when(e > 0)
    def _():
        o_ref[...] = o_ref[...] + contrib


@jax.jit
def kernel(x, gate_w, w1, w2):
    B, N, D = x.shape
    T = B * N
    xf = x.reshape(T, D)
    out = pl.pallas_call(
        _moe_body,
        grid=(E,),
        in_specs=[
            pl.BlockSpec((T, D), lambda e: (0, 0)),
            pl.BlockSpec((E, D), lambda e: (0, 0)),
            pl.BlockSpec((1, HID, D), lambda e: (e, 0, 0)),
            pl.BlockSpec((1, D, HID), lambda e: (e, 0, 0)),
        ],
        out_specs=pl.BlockSpec((T, D), lambda e: (0, 0)),
        out_shape=jax.ShapeDtypeStruct((T, D), jnp.float32),
    )(xf, gate_w, w1, w2)
    return out.reshape(B, N, D)
